# baseline (speedup 1.0000x reference)
import jax
import jax.numpy as jnp
from jax import lax
from jax.experimental import pallas as pl
from jax.experimental.pallas import tpu as pltpu
from jax.experimental.pallas import tpu_sc as plsc

N = 10000
E = 320000
D = 128
H = 8
DH = 16
AW = 136
CH = 64
NCHUNK = E // CH
NWORK = 32
ROWS_PER_SUB = N // 16
TCB = 10000

_HI = jax.lax.Precision.HIGHEST
_PR = jax.lax.Precision.DEFAULT


def _tc_head(x, W, C, P):
    def body(x_ref, w_ref, c_ref, p_ref, fb_ref, e_ref):
        f = jnp.dot(x_ref[...], w_ref[...], preferred_element_type=jnp.float32,
                    precision=_PR)
        fp = jnp.dot(f, p_ref[...], preferred_element_type=jnp.float32,
                     precision=_PR)
        fb_ref[...] = fp.astype(jnp.bfloat16)
        e_ref[...] = jnp.dot(f, c_ref[...], preferred_element_type=jnp.float32,
                             precision=_PR)

    return pl.pallas_call(
        body,
        grid=(N // TCB,),
        in_specs=[
            pl.BlockSpec((TCB, D), lambda i: (i, 0)),
            pl.BlockSpec((D, D), lambda i: (0, 0)),
            pl.BlockSpec((D, 16), lambda i: (0, 0)),
            pl.BlockSpec((D, D), lambda i: (0, 0)),
        ],
        out_specs=[
            pl.BlockSpec((TCB, D), lambda i: (i, 0)),
            pl.BlockSpec((TCB, 16), lambda i: (i, 0)),
        ],
        out_shape=[
            jax.ShapeDtypeStruct((N, D), jnp.bfloat16),
            jax.ShapeDtypeStruct((N, 16), jnp.float32),
        ],
    )(x, W, C, P)


def _tc_mid(acc, b, W, C, R, P):
    def body(a_ref, b_ref, w_ref, c_ref, r_ref, p_ref, fb_ref, e_ref):
        num = a_ref[0, :, :D] + a_ref[1, :, :D]
        den8 = a_ref[0, :, D:D + H] + a_ref[1, :, D:D + H]
        den = jnp.dot(den8, r_ref[...], preferred_element_type=jnp.float32,
                      precision=_HI)
        pre = jnp.where(den > 0.0, num / den, 0.0) + b_ref[...]
        hact = jnp.where(pre >= 0.0, pre, 0.01 * pre)
        f = jnp.dot(hact, w_ref[...], preferred_element_type=jnp.float32,
                    precision=_PR)
        fp = jnp.dot(f, p_ref[...], preferred_element_type=jnp.float32,
                     precision=_PR)
        fb_ref[...] = fp.astype(jnp.bfloat16)
        e_ref[...] = jnp.dot(f, c_ref[...], preferred_element_type=jnp.float32,
                             precision=_PR)

    return pl.pallas_call(
        body,
        grid=(N // TCB,),
        in_specs=[
            pl.BlockSpec((2, TCB, AW), lambda i: (0, i, 0)),
            pl.BlockSpec((1, D), lambda i: (0, 0)),
            pl.BlockSpec((D, D), lambda i: (0, 0)),
            pl.BlockSpec((D, 16), lambda i: (0, 0)),
            pl.BlockSpec((H, D), lambda i: (0, 0)),
            pl.BlockSpec((D, D), lambda i: (0, 0)),
        ],
        out_specs=[
            pl.BlockSpec((TCB, D), lambda i: (i, 0)),
            pl.BlockSpec((TCB, 16), lambda i: (i, 0)),
        ],
        out_shape=[
            jax.ShapeDtypeStruct((N, D), jnp.bfloat16),
            jax.ShapeDtypeStruct((N, 16), jnp.float32),
        ],
    )(acc, b, W, C, R, P)


def _tc_tail(acc, b, R):
    def body(a_ref, b_ref, r_ref, o_ref):
        num = a_ref[0, :, :D] + a_ref[1, :, :D]
        den8 = a_ref[0, :, D:D + H] + a_ref[1, :, D:D + H]
        den = jnp.dot(den8, r_ref[...], preferred_element_type=jnp.float32,
                      precision=_HI)
        o_ref[...] = jnp.where(den > 0.0, num / den, 0.0) + b_ref[...]

    return pl.pallas_call(
        body,
        grid=(N // TCB,),
        in_specs=[
            pl.BlockSpec((2, TCB, AW), lambda i: (0, i, 0)),
            pl.BlockSpec((1, D), lambda i: (0, 0)),
            pl.BlockSpec((H, D), lambda i: (0, 0)),
        ],
        out_specs=pl.BlockSpec((TCB, D), lambda i: (i, 0)),
        out_shape=jax.ShapeDtypeStruct((N, D), jnp.float32),
    )(acc, b, R)


def _sc_edge_pass(feat, eler, src, dst):
    mesh = plsc.VectorSubcoreMesh(core_axis_name="c", subcore_axis_name="s")
    NCH = NCHUNK // NWORK
    NREM = NCHUNK - NCH * NWORK

    def body(feat_hbm, eler_hbm, src_hbm, dst_hbm, out_hbm, acc,
             sidx0, sidx1, didx0, didx1, dscat0, dscat1,
             gs0, gs1, gd0, gd1, gb0, gb1, rows0, rows1,
             isem, gsem, ssem):
        SIDX = (sidx0, sidx1)
        DIDX = (didx0, didx1)
        DSCAT = (dscat0, dscat1)
        GS = (gs0, gs1)
        GD = (gd0, gd1)
        GB = (gb0, gb1)
        ROWS = (rows0, rows1)
        c = lax.axis_index("c")
        s = lax.axis_index("s")
        wid = s * 2 + c

        def issue_idx(ci, b):
            base = (wid + NWORK * ci) * CH
            pltpu.async_copy(src_hbm.at[pl.ds(base, CH)], SIDX[b],
                             isem.at[b])
            pltpu.async_copy(dst_hbm.at[pl.ds(base, CH)], DIDX[b],
                             isem.at[b])

        def wait_idx(b):
            pltpu.make_async_copy(src_hbm.at[pl.ds(0, CH)], SIDX[b],
                                  isem.at[b]).wait()
            pltpu.make_async_copy(dst_hbm.at[pl.ds(0, CH)], DIDX[b],
                                  isem.at[b]).wait()

        def issue_gathers(b):
            pltpu.async_copy(eler_hbm.at[SIDX[b]], GS[b], gsem.at[b])
            pltpu.async_copy(eler_hbm.at[DIDX[b]], GD[b], gsem.at[b])
            pltpu.async_copy(feat_hbm.at[SIDX[b]], GB[b], gsem.at[b])

        def wait_gathers(b):
            pltpu.make_async_copy(eler_hbm.at[SIDX[b]], GS[b],
                                  gsem.at[b]).wait()
            pltpu.make_async_copy(eler_hbm.at[DIDX[b]], GD[b],
                                  gsem.at[b]).wait()
            pltpu.make_async_copy(feat_hbm.at[SIDX[b]], GB[b],
                                  gsem.at[b]).wait()

        def issue_scatter(b):
            pltpu.async_copy(ROWS[b], acc.at[DSCAT[b]], ssem.at[b],
                             add=True)

        def wait_scatter(b):
            pltpu.make_async_copy(ROWS[b], acc.at[DSCAT[b]],
                                  ssem.at[b]).wait()

        iota16 = lax.iota(jnp.int32, 16)
        wcol = D + (iota16 & 7)

        def save_didx(b):
            for i in range(CH // 16):
                DSCAT[b][pl.ds(i * 16, 16)] = DIDX[b][pl.ds(i * 16, 16)]

        def compute(b):
            @plsc.parallel_loop(0, CH * H // 16, unroll=4)
            def _wloop(i):
                p = i * 16 + iota16
                k = p >> 3
                h = p & 7
                elv = plsc.load_gather(GS[b], [k, h])
                erv = plsc.load_gather(GD[b], [k, h + 8])
                sv = elv + erv
                w = jnp.exp(jnp.maximum(sv, 0.2 * sv))
                plsc.store_scatter(ROWS[b], [k, h + D], w)

            @plsc.parallel_loop(0, CH, unroll=2)
            def _sloop(k):
                wv = plsc.load_gather(ROWS[b], [jnp.full((16,), k, jnp.int32),
                                                wcol])
                for q in range(H // 2):
                    x = GB[b][k, pl.ds(32 * q, 32)]
                    va, vb = plsc.unpack(x, format=plsc.PackFormat.INTERLEAVED)
                    ROWS[b][k, pl.ds(32 * q, DH)] = va * wv[2 * q]
                    ROWS[b][k, pl.ds(32 * q + DH, DH)] = vb * wv[2 * q + 1]

        zero16 = jnp.zeros((16,), jnp.float32)
        for b in (0, 1):
            @pl.loop(0, CH)
            def _zrow(k):
                @pl.loop(0, AW, step=16)
                def _zcol(j):
                    ROWS[b][k, pl.ds(j, 16)] = zero16

        zbase = s * ROWS_PER_SUB
        for j in range(ROWS_PER_SUB // CH):
            pltpu.sync_copy(rows0,
                            acc.at[pl.ds(zbase + CH * j, CH)])
        _tail = ROWS_PER_SUB % CH
        if _tail:
            pltpu.sync_copy(rows0.at[pl.ds(0, _tail)],
                            acc.at[pl.ds(zbase + ROWS_PER_SUB - _tail, _tail)])
        plsc.subcore_barrier()

        issue_idx(0, 0)
        issue_idx(1, 1)
        wait_idx(0)
        issue_gathers(0)

        @pl.loop(0, NCH, step=2)
        def _chunks(t):
            for b in (0, 1):
                tt = t + b
                nb = 1 - b
                wait_gathers(b)

                @pl.when(tt >= 2)
                def _(b=b):
                    wait_scatter(b)

                save_didx(b)

                @pl.when(tt + 2 < NCH)
                def _(tt=tt, b=b):
                    issue_idx(tt + 2, b)

                @pl.when(tt + 1 < NCH)
                def _(b=b, nb=nb):
                    wait_idx(nb)
                    issue_gathers(nb)

                compute(b)
                issue_scatter(b)

        wait_scatter(0)
        wait_scatter(1)

        @pl.when(wid < NREM)
        def _rem():
            base = (NCH * NWORK + wid) * CH
            pltpu.sync_copy(src_hbm.at[pl.ds(base, CH)], sidx0)
            pltpu.sync_copy(dst_hbm.at[pl.ds(base, CH)], didx0)
            issue_gathers(0)
            wait_gathers(0)
            save_didx(0)
            compute(0)
            issue_scatter(0)
            wait_scatter(0)

        plsc.subcore_barrier()

        rbase = s * ROWS_PER_SUB
        pltpu.sync_copy(acc.at[pl.ds(rbase, ROWS_PER_SUB)],
                        out_hbm.at[c, pl.ds(rbase, ROWS_PER_SUB)])

    kern = pl.kernel(
        body,
        out_type=jax.ShapeDtypeStruct((2, N, AW), jnp.float32),
        mesh=mesh,
        compiler_params=pltpu.CompilerParams(use_tc_tiling_on_sc=False,
                                             needs_layout_passes=False),
        scratch_types=[
            pltpu.VMEM_SHARED((N, AW), jnp.float32),
            pltpu.VMEM((CH,), jnp.int32),
            pltpu.VMEM((CH,), jnp.int32),
            pltpu.VMEM((CH,), jnp.int32),
            pltpu.VMEM((CH,), jnp.int32),
            pltpu.VMEM((CH,), jnp.int32),
            pltpu.VMEM((CH,), jnp.int32),
            pltpu.VMEM((CH, 16), jnp.float32),
            pltpu.VMEM((CH, 16), jnp.float32),
            pltpu.VMEM((CH, 16), jnp.float32),
            pltpu.VMEM((CH, 16), jnp.float32),
            pltpu.VMEM((CH, D), jnp.bfloat16),
            pltpu.VMEM((CH, D), jnp.bfloat16),
            pltpu.VMEM((CH, AW), jnp.float32),
            pltpu.VMEM((CH, AW), jnp.float32),
            pltpu.SemaphoreType.DMA((2,)),
            pltpu.SemaphoreType.DMA((2,)),
            pltpu.SemaphoreType.DMA((2,)),
        ],
    )
    return kern(feat, eler, src, dst)


def _mix_matrix(al, ar):
    rows = jnp.arange(D)
    h = rows // DH
    j = rows % DH
    C = jnp.zeros((D, 2 * H), jnp.float32)
    C = C.at[rows, h].set(al[h, j])
    C = C.at[rows, H + h].set(ar[h, j])
    return C


def _perm_matrix():
    i = jnp.arange(D)
    h = i // DH
    j = i % DH
    dcol = 32 * (h // 2) + 2 * j + (h % 2)
    return (jnp.arange(D)[None, :] == dcol[:, None]).astype(jnp.float32)


def _rep_matrix():
    cols = jnp.arange(D)
    return (jnp.arange(H)[:, None] == (cols[None, :] // DH)).astype(jnp.float32)


def kernel(n_feat, edge_index, W0, al0, ar0, b0, W1, al1, ar1, b1):
    src = edge_index[0].astype(jnp.int32)
    dst = edge_index[1].astype(jnp.int32)
    C0 = _mix_matrix(al0, ar0)
    C1 = _mix_matrix(al1, ar1)
    R = _rep_matrix()
    P = _perm_matrix()
    b0r = b0.reshape(1, D)
    b1r = b1.reshape(1, D)

    featb0, eler0 = _tc_head(n_feat, W0, C0, P)
    acc0 = _sc_edge_pass(featb0, eler0, src, dst)
    featb1, eler1 = _tc_mid(acc0, b0r, W1, C1, R, P)
    acc1 = _sc_edge_pass(featb1, eler1, src, dst)
    return _tc_tail(acc1, b1r, R)


# speedup vs baseline: 1.0123x; 1.0123x over previous
import jax
import jax.numpy as jnp
from jax import lax
from jax.experimental import pallas as pl
from jax.experimental.pallas import tpu as pltpu
from jax.experimental.pallas import tpu_sc as plsc

N = 10000
E = 320000
D = 128
H = 8
DH = 16
AW = 136
CH = 64
NCHUNK = E // CH
NWORK = 32
ROWS_PER_SUB = N // 16
TCB = 2000

_HI = jax.lax.Precision.HIGHEST
_PR = jax.lax.Precision.DEFAULT


def _tc_head(x, W, C, P):
    def body(x_ref, w_ref, c_ref, p_ref, fb_ref, e_ref):
        f = jnp.dot(x_ref[...], w_ref[...], preferred_element_type=jnp.float32,
                    precision=_PR)
        fp = jnp.dot(f, p_ref[...], preferred_element_type=jnp.float32,
                     precision=_PR)
        fb_ref[...] = fp.astype(jnp.bfloat16)
        e_ref[...] = jnp.dot(f, c_ref[...], preferred_element_type=jnp.float32,
                             precision=_PR)

    return pl.pallas_call(
        body,
        grid=(N // TCB,),
        in_specs=[
            pl.BlockSpec((TCB, D), lambda i: (i, 0)),
            pl.BlockSpec((D, D), lambda i: (0, 0)),
            pl.BlockSpec((D, 16), lambda i: (0, 0)),
            pl.BlockSpec((D, D), lambda i: (0, 0)),
        ],
        out_specs=[
            pl.BlockSpec((TCB, D), lambda i: (i, 0)),
            pl.BlockSpec((TCB, 16), lambda i: (i, 0)),
        ],
        out_shape=[
            jax.ShapeDtypeStruct((N, D), jnp.bfloat16),
            jax.ShapeDtypeStruct((N, 16), jnp.float32),
        ],
    )(x, W, C, P)


def _tc_mid(acc, b, W, C, R, P):
    def body(a_ref, b_ref, w_ref, c_ref, r_ref, p_ref, fb_ref, e_ref):
        num = a_ref[0, :, :D] + a_ref[1, :, :D]
        den8 = a_ref[0, :, D:D + H] + a_ref[1, :, D:D + H]
        den = jnp.dot(den8, r_ref[...], preferred_element_type=jnp.float32,
                      precision=_HI)
        pre = jnp.where(den > 0.0, num / den, 0.0) + b_ref[...]
        hact = jnp.where(pre >= 0.0, pre, 0.01 * pre)
        f = jnp.dot(hact, w_ref[...], preferred_element_type=jnp.float32,
                    precision=_PR)
        fp = jnp.dot(f, p_ref[...], preferred_element_type=jnp.float32,
                     precision=_PR)
        fb_ref[...] = fp.astype(jnp.bfloat16)
        e_ref[...] = jnp.dot(f, c_ref[...], preferred_element_type=jnp.float32,
                             precision=_PR)

    return pl.pallas_call(
        body,
        grid=(N // TCB,),
        in_specs=[
            pl.BlockSpec((2, TCB, AW), lambda i: (0, i, 0)),
            pl.BlockSpec((1, D), lambda i: (0, 0)),
            pl.BlockSpec((D, D), lambda i: (0, 0)),
            pl.BlockSpec((D, 16), lambda i: (0, 0)),
            pl.BlockSpec((H, D), lambda i: (0, 0)),
            pl.BlockSpec((D, D), lambda i: (0, 0)),
        ],
        out_specs=[
            pl.BlockSpec((TCB, D), lambda i: (i, 0)),
            pl.BlockSpec((TCB, 16), lambda i: (i, 0)),
        ],
        out_shape=[
            jax.ShapeDtypeStruct((N, D), jnp.bfloat16),
            jax.ShapeDtypeStruct((N, 16), jnp.float32),
        ],
    )(acc, b, W, C, R, P)


def _tc_tail(acc, b, R):
    def body(a_ref, b_ref, r_ref, o_ref):
        num = a_ref[0, :, :D] + a_ref[1, :, :D]
        den8 = a_ref[0, :, D:D + H] + a_ref[1, :, D:D + H]
        den = jnp.dot(den8, r_ref[...], preferred_element_type=jnp.float32,
                      precision=_HI)
        o_ref[...] = jnp.where(den > 0.0, num / den, 0.0) + b_ref[...]

    return pl.pallas_call(
        body,
        grid=(N // TCB,),
        in_specs=[
            pl.BlockSpec((2, TCB, AW), lambda i: (0, i, 0)),
            pl.BlockSpec((1, D), lambda i: (0, 0)),
            pl.BlockSpec((H, D), lambda i: (0, 0)),
        ],
        out_specs=pl.BlockSpec((TCB, D), lambda i: (i, 0)),
        out_shape=jax.ShapeDtypeStruct((N, D), jnp.float32),
    )(acc, b, R)


def _sc_edge_pass(feat, eler, src, dst):
    mesh = plsc.VectorSubcoreMesh(core_axis_name="c", subcore_axis_name="s")
    NCH = NCHUNK // NWORK
    NREM = NCHUNK - NCH * NWORK

    def body(feat_hbm, eler_hbm, src_hbm, dst_hbm, out_hbm, acc,
             sd0, sd1, dscat0, dscat1,
             gde0, gde1, gb0, gb1, rows0, rows1,
             isem, gsem, ssem):
        SD = (sd0, sd1)
        DSCAT = (dscat0, dscat1)
        GDE = (gde0, gde1)
        GB = (gb0, gb1)
        ROWS = (rows0, rows1)
        c = lax.axis_index("c")
        s = lax.axis_index("s")
        wid = s * 2 + c

        def issue_idx(ci, b):
            base = (wid + NWORK * ci) * CH
            pltpu.async_copy(src_hbm.at[pl.ds(base, CH)],
                             SD[b].at[pl.ds(0, CH)], isem.at[b])
            pltpu.async_copy(dst_hbm.at[pl.ds(base, CH)],
                             SD[b].at[pl.ds(CH, CH)], isem.at[b])

        def wait_idx(b):
            pltpu.make_async_copy(src_hbm.at[pl.ds(0, CH)],
                                  SD[b].at[pl.ds(0, CH)], isem.at[b]).wait()
            pltpu.make_async_copy(dst_hbm.at[pl.ds(0, CH)],
                                  SD[b].at[pl.ds(CH, CH)], isem.at[b]).wait()

        def issue_gathers(b):
            pltpu.async_copy(eler_hbm.at[SD[b]], GDE[b], gsem.at[b])
            pltpu.async_copy(feat_hbm.at[SD[b].at[pl.ds(0, CH)]], GB[b],
                             gsem.at[b])

        def wait_gathers(b):
            pltpu.make_async_copy(eler_hbm.at[SD[b]], GDE[b],
                                  gsem.at[b]).wait()
            pltpu.make_async_copy(feat_hbm.at[SD[b].at[pl.ds(0, CH)]], GB[b],
                                  gsem.at[b]).wait()

        def issue_scatter(b):
            pltpu.async_copy(ROWS[b], acc.at[DSCAT[b]], ssem.at[b],
                             add=True)

        def wait_scatter(b):
            pltpu.make_async_copy(ROWS[b], acc.at[DSCAT[b]],
                                  ssem.at[b]).wait()

        iota16 = lax.iota(jnp.int32, 16)
        wcol = D + (iota16 & 7)

        def save_didx(b):
            for i in range(CH // 16):
                DSCAT[b][pl.ds(i * 16, 16)] = SD[b][pl.ds(CH + i * 16, 16)]

        def compute(b):
            @plsc.parallel_loop(0, CH * H // 16, unroll=4)
            def _wloop(i):
                p = i * 16 + iota16
                k = p >> 3
                h = p & 7
                elv = plsc.load_gather(GDE[b], [k, h])
                erv = plsc.load_gather(GDE[b], [k + CH, h + 8])
                sv = elv + erv
                w = jnp.exp(jnp.maximum(sv, 0.2 * sv))
                plsc.store_scatter(ROWS[b], [k, h + D], w)

            @plsc.parallel_loop(0, CH, unroll=2)
            def _sloop(k):
                wv = plsc.load_gather(ROWS[b], [jnp.full((16,), k, jnp.int32),
                                                wcol])
                for q in range(H // 2):
                    x = GB[b][k, pl.ds(32 * q, 32)]
                    va, vb = plsc.unpack(x, format=plsc.PackFormat.INTERLEAVED)
                    ROWS[b][k, pl.ds(32 * q, DH)] = va * wv[2 * q]
                    ROWS[b][k, pl.ds(32 * q + DH, DH)] = vb * wv[2 * q + 1]

        zero16 = jnp.zeros((16,), jnp.float32)
        for b in (0, 1):
            @pl.loop(0, CH)
            def _zrow(k):
                @pl.loop(0, AW, step=16)
                def _zcol(j):
                    ROWS[b][k, pl.ds(j, 16)] = zero16

        zbase = s * ROWS_PER_SUB
        for j in range(ROWS_PER_SUB // CH):
            pltpu.sync_copy(rows0,
                            acc.at[pl.ds(zbase + CH * j, CH)])
        _tail = ROWS_PER_SUB % CH
        if _tail:
            pltpu.sync_copy(rows0.at[pl.ds(0, _tail)],
                            acc.at[pl.ds(zbase + ROWS_PER_SUB - _tail, _tail)])
        plsc.subcore_barrier()

        issue_idx(0, 0)
        issue_idx(1, 1)
        wait_idx(0)
        issue_gathers(0)

        @pl.loop(0, NCH, step=2)
        def _chunks(t):
            for b in (0, 1):
                tt = t + b
                nb = 1 - b
                wait_gathers(b)

                @pl.when(tt >= 2)
                def _(b=b):
                    wait_scatter(b)

                save_didx(b)

                @pl.when(tt + 2 < NCH)
                def _(tt=tt, b=b):
                    issue_idx(tt + 2, b)

                @pl.when(tt + 1 < NCH)
                def _(b=b, nb=nb):
                    wait_idx(nb)
                    issue_gathers(nb)

                compute(b)
                issue_scatter(b)

        wait_scatter(0)
        wait_scatter(1)

        @pl.when(wid < NREM)
        def _rem():
            base = (NCH * NWORK + wid) * CH
            pltpu.sync_copy(src_hbm.at[pl.ds(base, CH)],
                            sd0.at[pl.ds(0, CH)])
            pltpu.sync_copy(dst_hbm.at[pl.ds(base, CH)],
                            sd0.at[pl.ds(CH, CH)])
            issue_gathers(0)
            wait_gathers(0)
            save_didx(0)
            compute(0)
            issue_scatter(0)
            wait_scatter(0)

        plsc.subcore_barrier()

        rbase = s * ROWS_PER_SUB
        pltpu.sync_copy(acc.at[pl.ds(rbase, ROWS_PER_SUB)],
                        out_hbm.at[c, pl.ds(rbase, ROWS_PER_SUB)])

    kern = pl.kernel(
        body,
        out_type=jax.ShapeDtypeStruct((2, N, AW), jnp.float32),
        mesh=mesh,
        compiler_params=pltpu.CompilerParams(use_tc_tiling_on_sc=False,
                                             needs_layout_passes=False),
        scratch_types=[
            pltpu.VMEM_SHARED((N, AW), jnp.float32),
            pltpu.VMEM((2 * CH,), jnp.int32),
            pltpu.VMEM((2 * CH,), jnp.int32),
            pltpu.VMEM((CH,), jnp.int32),
            pltpu.VMEM((CH,), jnp.int32),
            pltpu.VMEM((2 * CH, 16), jnp.float32),
            pltpu.VMEM((2 * CH, 16), jnp.float32),
            pltpu.VMEM((CH, D), jnp.bfloat16),
            pltpu.VMEM((CH, D), jnp.bfloat16),
            pltpu.VMEM((CH, AW), jnp.float32),
            pltpu.VMEM((CH, AW), jnp.float32),
            pltpu.SemaphoreType.DMA((2,)),
            pltpu.SemaphoreType.DMA((2,)),
            pltpu.SemaphoreType.DMA((2,)),
        ],
    )
    return kern(feat, eler, src, dst)


def _mix_matrix(al, ar):
    rows = jnp.arange(D)
    h = rows // DH
    j = rows % DH
    C = jnp.zeros((D, 2 * H), jnp.float32)
    C = C.at[rows, h].set(al[h, j])
    C = C.at[rows, H + h].set(ar[h, j])
    return C


def _perm_matrix():
    i = jnp.arange(D)
    h = i // DH
    j = i % DH
    dcol = 32 * (h // 2) + 2 * j + (h % 2)
    return (jnp.arange(D)[None, :] == dcol[:, None]).astype(jnp.float32)


def _rep_matrix():
    cols = jnp.arange(D)
    return (jnp.arange(H)[:, None] == (cols[None, :] // DH)).astype(jnp.float32)


def kernel(n_feat, edge_index, W0, al0, ar0, b0, W1, al1, ar1, b1):
    src = edge_index[0].astype(jnp.int32)
    dst = edge_index[1].astype(jnp.int32)
    C0 = _mix_matrix(al0, ar0)
    C1 = _mix_matrix(al1, ar1)
    R = _rep_matrix()
    P = _perm_matrix()
    b0r = b0.reshape(1, D)
    b1r = b1.reshape(1, D)

    featb0, eler0 = _tc_head(n_feat, W0, C0, P)
    acc0 = _sc_edge_pass(featb0, eler0, src, dst)
    featb1, eler1 = _tc_mid(acc0, b0r, W1, C1, R, P)
    acc1 = _sc_edge_pass(featb1, eler1, src, dst)
    return _tc_tail(acc1, b1r, R)


# speedup vs baseline: 1.0126x; 1.0003x over previous
import jax
import jax.numpy as jnp
from jax import lax
from jax.experimental import pallas as pl
from jax.experimental.pallas import tpu as pltpu
from jax.experimental.pallas import tpu_sc as plsc

N = 10000
E = 320000
D = 128
H = 8
DH = 16
AW = 136
CH = 64
NCHUNK = E // CH
NWORK = 32
ROWS_PER_SUB = N // 16
TCB = 2000

_HI = jax.lax.Precision.HIGHEST
_PR = jax.lax.Precision.DEFAULT


def _tc_head(x, W, C, P):
    def body(x_ref, w_ref, c_ref, p_ref, fb_ref, e_ref):
        f = jnp.dot(x_ref[...], w_ref[...], preferred_element_type=jnp.float32,
                    precision=_PR)
        fp = jnp.dot(f, p_ref[...], preferred_element_type=jnp.float32,
                     precision=_PR)
        fb_ref[...] = fp.astype(jnp.bfloat16)
        e_ref[...] = jnp.dot(f, c_ref[...], preferred_element_type=jnp.float32,
                             precision=_PR)

    return pl.pallas_call(
        body,
        grid=(N // TCB,),
        in_specs=[
            pl.BlockSpec((TCB, D), lambda i: (i, 0)),
            pl.BlockSpec((D, D), lambda i: (0, 0)),
            pl.BlockSpec((D, 16), lambda i: (0, 0)),
            pl.BlockSpec((D, D), lambda i: (0, 0)),
        ],
        out_specs=[
            pl.BlockSpec((TCB, D), lambda i: (i, 0)),
            pl.BlockSpec((TCB, 16), lambda i: (i, 0)),
        ],
        out_shape=[
            jax.ShapeDtypeStruct((N, D), jnp.bfloat16),
            jax.ShapeDtypeStruct((N, 16), jnp.float32),
        ],
    )(x, W, C, P)


def _tc_mid(acc, b, W, C, R, P):
    def body(a_ref, b_ref, w_ref, c_ref, r_ref, p_ref, fb_ref, e_ref):
        num = a_ref[0, :, :D] + a_ref[1, :, :D]
        den8 = a_ref[0, :, D:D + H] + a_ref[1, :, D:D + H]
        den = jnp.dot(den8, r_ref[...], preferred_element_type=jnp.float32,
                      precision=_HI)
        pre = jnp.where(den > 0.0, num / den, 0.0) + b_ref[...]
        hact = jnp.where(pre >= 0.0, pre, 0.01 * pre)
        f = jnp.dot(hact, w_ref[...], preferred_element_type=jnp.float32,
                    precision=_PR)
        fp = jnp.dot(f, p_ref[...], preferred_element_type=jnp.float32,
                     precision=_PR)
        fb_ref[...] = fp.astype(jnp.bfloat16)
        e_ref[...] = jnp.dot(f, c_ref[...], preferred_element_type=jnp.float32,
                             precision=_PR)

    return pl.pallas_call(
        body,
        grid=(N // TCB,),
        in_specs=[
            pl.BlockSpec((2, TCB, AW), lambda i: (0, i, 0)),
            pl.BlockSpec((1, D), lambda i: (0, 0)),
            pl.BlockSpec((D, D), lambda i: (0, 0)),
            pl.BlockSpec((D, 16), lambda i: (0, 0)),
            pl.BlockSpec((H, D), lambda i: (0, 0)),
            pl.BlockSpec((D, D), lambda i: (0, 0)),
        ],
        out_specs=[
            pl.BlockSpec((TCB, D), lambda i: (i, 0)),
            pl.BlockSpec((TCB, 16), lambda i: (i, 0)),
        ],
        out_shape=[
            jax.ShapeDtypeStruct((N, D), jnp.bfloat16),
            jax.ShapeDtypeStruct((N, 16), jnp.float32),
        ],
    )(acc, b, W, C, R, P)


def _tc_tail(acc, b, R):
    def body(a_ref, b_ref, r_ref, o_ref):
        num = a_ref[0, :, :D] + a_ref[1, :, :D]
        den8 = a_ref[0, :, D:D + H] + a_ref[1, :, D:D + H]
        den = jnp.dot(den8, r_ref[...], preferred_element_type=jnp.float32,
                      precision=_HI)
        o_ref[...] = jnp.where(den > 0.0, num / den, 0.0) + b_ref[...]

    return pl.pallas_call(
        body,
        grid=(N // TCB,),
        in_specs=[
            pl.BlockSpec((2, TCB, AW), lambda i: (0, i, 0)),
            pl.BlockSpec((1, D), lambda i: (0, 0)),
            pl.BlockSpec((H, D), lambda i: (0, 0)),
        ],
        out_specs=pl.BlockSpec((TCB, D), lambda i: (i, 0)),
        out_shape=jax.ShapeDtypeStruct((N, D), jnp.float32),
    )(acc, b, R)


def _sc_edge_pass(feat, eler, src, dst):
    mesh = plsc.VectorSubcoreMesh(core_axis_name="c", subcore_axis_name="s")
    NCH = NCHUNK // NWORK
    NREM = NCHUNK - NCH * NWORK

    def body(feat_hbm, eler_hbm, src_hbm, dst_hbm, out_hbm, acc,
             sd0, sd1, dscat0, dscat1,
             gde0, gde1, gb0, gb1, rows0, rows1,
             isem, gsem, ssem):
        SD = (sd0, sd1)
        DSCAT = (dscat0, dscat1)
        GDE = (gde0, gde1)
        GB = (gb0, gb1)
        ROWS = (rows0, rows1)
        c = lax.axis_index("c")
        s = lax.axis_index("s")
        wid = s * 2 + c

        def issue_idx(ci, b):
            base = (wid + NWORK * ci) * CH
            pltpu.async_copy(src_hbm.at[pl.ds(base, CH)],
                             SD[b].at[pl.ds(0, CH)], isem.at[b])
            pltpu.async_copy(dst_hbm.at[pl.ds(base, CH)],
                             SD[b].at[pl.ds(CH, CH)], isem.at[b])

        def wait_idx(b):
            pltpu.make_async_copy(src_hbm.at[pl.ds(0, CH)],
                                  SD[b].at[pl.ds(0, CH)], isem.at[b]).wait()
            pltpu.make_async_copy(dst_hbm.at[pl.ds(0, CH)],
                                  SD[b].at[pl.ds(CH, CH)], isem.at[b]).wait()

        def issue_gathers(b):
            pltpu.async_copy(eler_hbm.at[SD[b]], GDE[b], gsem.at[b])
            pltpu.async_copy(feat_hbm.at[SD[b].at[pl.ds(0, CH)]], GB[b],
                             gsem.at[b])

        def wait_gathers(b):
            pltpu.make_async_copy(eler_hbm.at[SD[b]], GDE[b],
                                  gsem.at[b]).wait()
            pltpu.make_async_copy(feat_hbm.at[SD[b].at[pl.ds(0, CH)]], GB[b],
                                  gsem.at[b]).wait()

        def issue_scatter(b):
            pltpu.async_copy(ROWS[b], acc.at[DSCAT[b]], ssem.at[b],
                             add=True)

        def wait_scatter(b):
            pltpu.make_async_copy(ROWS[b], acc.at[DSCAT[b]],
                                  ssem.at[b]).wait()

        iota16 = lax.iota(jnp.int32, 16)
        wcol = D + (iota16 & 7)

        def save_didx(b):
            for i in range(CH // 16):
                DSCAT[b][pl.ds(i * 16, 16)] = SD[b][pl.ds(CH + i * 16, 16)]

        def compute(b):
            @plsc.parallel_loop(0, CH * H // 16, unroll=8)
            def _wloop(i):
                p = i * 16 + iota16
                k = p >> 3
                h = p & 7
                elv = plsc.load_gather(GDE[b], [k, h])
                erv = plsc.load_gather(GDE[b], [k + CH, h + 8])
                sv = elv + erv
                w = jnp.exp(jnp.maximum(sv, 0.2 * sv))
                plsc.store_scatter(ROWS[b], [k, h + D], w)

            @plsc.parallel_loop(0, CH, unroll=4)
            def _sloop(k):
                wv = plsc.load_gather(ROWS[b], [jnp.full((16,), k, jnp.int32),
                                                wcol])
                for q in range(H // 2):
                    x = GB[b][k, pl.ds(32 * q, 32)]
                    va, vb = plsc.unpack(x, format=plsc.PackFormat.INTERLEAVED)
                    ROWS[b][k, pl.ds(32 * q, DH)] = va * wv[2 * q]
                    ROWS[b][k, pl.ds(32 * q + DH, DH)] = vb * wv[2 * q + 1]

        zero16 = jnp.zeros((16,), jnp.float32)
        for b in (0, 1):
            @pl.loop(0, CH)
            def _zrow(k):
                @pl.loop(0, AW, step=16)
                def _zcol(j):
                    ROWS[b][k, pl.ds(j, 16)] = zero16

        zbase = s * ROWS_PER_SUB
        for j in range(ROWS_PER_SUB // CH):
            pltpu.sync_copy(rows0,
                            acc.at[pl.ds(zbase + CH * j, CH)])
        _tail = ROWS_PER_SUB % CH
        if _tail:
            pltpu.sync_copy(rows0.at[pl.ds(0, _tail)],
                            acc.at[pl.ds(zbase + ROWS_PER_SUB - _tail, _tail)])
        plsc.subcore_barrier()

        issue_idx(0, 0)
        issue_idx(1, 1)
        wait_idx(0)
        issue_gathers(0)

        @pl.loop(0, NCH, step=2)
        def _chunks(t):
            for b in (0, 1):
                tt = t + b
                nb = 1 - b
                wait_gathers(b)

                @pl.when(tt >= 2)
                def _(b=b):
                    wait_scatter(b)

                save_didx(b)

                @pl.when(tt + 2 < NCH)
                def _(tt=tt, b=b):
                    issue_idx(tt + 2, b)

                @pl.when(tt + 1 < NCH)
                def _(b=b, nb=nb):
                    wait_idx(nb)
                    issue_gathers(nb)

                compute(b)
                issue_scatter(b)

        wait_scatter(0)
        wait_scatter(1)

        @pl.when(wid < NREM)
        def _rem():
            base = (NCH * NWORK + wid) * CH
            pltpu.sync_copy(src_hbm.at[pl.ds(base, CH)],
                            sd0.at[pl.ds(0, CH)])
            pltpu.sync_copy(dst_hbm.at[pl.ds(base, CH)],
                            sd0.at[pl.ds(CH, CH)])
            issue_gathers(0)
            wait_gathers(0)
            save_didx(0)
            compute(0)
            issue_scatter(0)
            wait_scatter(0)

        plsc.subcore_barrier()

        rbase = s * ROWS_PER_SUB
        pltpu.sync_copy(acc.at[pl.ds(rbase, ROWS_PER_SUB)],
                        out_hbm.at[c, pl.ds(rbase, ROWS_PER_SUB)])

    kern = pl.kernel(
        body,
        out_type=jax.ShapeDtypeStruct((2, N, AW), jnp.float32),
        mesh=mesh,
        compiler_params=pltpu.CompilerParams(use_tc_tiling_on_sc=False,
                                             needs_layout_passes=False),
        scratch_types=[
            pltpu.VMEM_SHARED((N, AW), jnp.float32),
            pltpu.VMEM((2 * CH,), jnp.int32),
            pltpu.VMEM((2 * CH,), jnp.int32),
            pltpu.VMEM((CH,), jnp.int32),
            pltpu.VMEM((CH,), jnp.int32),
            pltpu.VMEM((2 * CH, 16), jnp.float32),
            pltpu.VMEM((2 * CH, 16), jnp.float32),
            pltpu.VMEM((CH, D), jnp.bfloat16),
            pltpu.VMEM((CH, D), jnp.bfloat16),
            pltpu.VMEM((CH, AW), jnp.float32),
            pltpu.VMEM((CH, AW), jnp.float32),
            pltpu.SemaphoreType.DMA((2,)),
            pltpu.SemaphoreType.DMA((2,)),
            pltpu.SemaphoreType.DMA((2,)),
        ],
    )
    return kern(feat, eler, src, dst)


def _mix_matrix(al, ar):
    rows = jnp.arange(D)
    h = rows // DH
    j = rows % DH
    C = jnp.zeros((D, 2 * H), jnp.float32)
    C = C.at[rows, h].set(al[h, j])
    C = C.at[rows, H + h].set(ar[h, j])
    return C


def _perm_matrix():
    i = jnp.arange(D)
    h = i // DH
    j = i % DH
    dcol = 32 * (h // 2) + 2 * j + (h % 2)
    return (jnp.arange(D)[None, :] == dcol[:, None]).astype(jnp.float32)


def _rep_matrix():
    cols = jnp.arange(D)
    return (jnp.arange(H)[:, None] == (cols[None, :] // DH)).astype(jnp.float32)


def kernel(n_feat, edge_index, W0, al0, ar0, b0, W1, al1, ar1, b1):
    src = edge_index[0].astype(jnp.int32)
    dst = edge_index[1].astype(jnp.int32)
    C0 = _mix_matrix(al0, ar0)
    C1 = _mix_matrix(al1, ar1)
    R = _rep_matrix()
    P = _perm_matrix()
    b0r = b0.reshape(1, D)
    b1r = b1.reshape(1, D)

    featb0, eler0 = _tc_head(n_feat, W0, C0, P)
    acc0 = _sc_edge_pass(featb0, eler0, src, dst)
    featb1, eler1 = _tc_mid(acc0, b0r, W1, C1, R, P)
    acc1 = _sc_edge_pass(featb1, eler1, src, dst)
    return _tc_tail(acc1, b1r, R)
